# baseline (speedup 1.0000x reference)
import functools

import jax
import jax.numpy as jnp
from jax import lax
from jax.experimental import pallas as pl
from jax.experimental.pallas import tpu as pltpu
from jax.experimental.pallas import tpu_sc as plsc

NC, NS, L = 2, 16, 16
NW = NC * NS
BLOCK_M = 2048


def _logits_block(x_ref, w_ref, lt_ref):
    lt_ref[...] = lax.dot_general(
        w_ref[...], x_ref[...], (((1,), (1,)), ((), ())),
        preferred_element_type=jnp.float32,
    )


def _logits_t(x, w):
    n, h = x.shape
    e = w.shape[0]
    return pl.pallas_call(
        _logits_block,
        grid=(n // BLOCK_M,),
        in_specs=[
            pl.BlockSpec((BLOCK_M, h), lambda i: (i, 0)),
            pl.BlockSpec((e, h), lambda i: (0, 0)),
        ],
        out_specs=pl.BlockSpec((e, BLOCK_M), lambda i: (0, i)),
        out_shape=jax.ShapeDtypeStruct((e, n), jnp.float32),
    )(x, w)


def _sc_top2(lt):
    e, n = lt.shape
    rpt = n // NW
    groups = rpt // L
    mesh = plsc.VectorSubcoreMesh(core_axis_name="c", subcore_axis_name="s")

    @functools.partial(
        pl.kernel,
        mesh=mesh,
        out_type=[
            jax.ShapeDtypeStruct((n,), jnp.int32),
            jax.ShapeDtypeStruct((n,), jnp.int32),
            jax.ShapeDtypeStruct((n,), jnp.float32),
            jax.ShapeDtypeStruct((n,), jnp.float32),
        ],
        scratch_types=[
            pltpu.VMEM((e, rpt), jnp.float32),
            pltpu.VMEM((rpt,), jnp.int32),
            pltpu.VMEM((rpt,), jnp.int32),
            pltpu.VMEM((rpt,), jnp.float32),
            pltpu.VMEM((rpt,), jnp.float32),
        ],
    )
    def k(lt_hbm, i1_hbm, i2_hbm, w1_hbm, w2_hbm, lt_v, i1_v, i2_v, w1_v, w2_v):
        wid = lax.axis_index("s") * NC + lax.axis_index("c")
        base = wid * rpt
        pltpu.sync_copy(lt_hbm.at[:, pl.ds(base, rpt)], lt_v)

        def group(g, carry):
            col = pl.multiple_of(g * L, L)
            m1 = jnp.full((L,), -jnp.inf, jnp.float32)
            m2 = jnp.full((L,), -jnp.inf, jnp.float32)
            i1 = jnp.zeros((L,), jnp.int32)
            i2 = jnp.zeros((L,), jnp.int32)
            for ei in range(e):
                v = lt_v[ei, pl.ds(col, L)]
                gt1 = v > m1
                gt2 = v > m2
                i2 = jnp.where(gt1, i1, jnp.where(gt2, ei, i2))
                m2 = jnp.where(gt1, m1, jnp.where(gt2, v, m2))
                i1 = jnp.where(gt1, ei, i1)
                m1 = jnp.where(gt1, v, m1)
            e2 = jnp.exp(m2 - m1)
            w1 = 1.0 / (1.0 + e2)
            i1_v[pl.ds(col, L)] = i1
            i2_v[pl.ds(col, L)] = i2
            w1_v[pl.ds(col, L)] = w1
            w2_v[pl.ds(col, L)] = 1.0 - w1
            return carry

        lax.fori_loop(0, groups, group, 0)
        pltpu.sync_copy(i1_v, i1_hbm.at[pl.ds(base, rpt)])
        pltpu.sync_copy(i2_v, i2_hbm.at[pl.ds(base, rpt)])
        pltpu.sync_copy(w1_v, w1_hbm.at[pl.ds(base, rpt)])
        pltpu.sync_copy(w2_v, w2_hbm.at[pl.ds(base, rpt)])

    return k(lt)


@jax.jit
def _gate(x, w):
    lt = _logits_t(x, w)
    i1, i2, w1, w2 = _sc_top2(lt)
    idx = jnp.stack([i1, i2], axis=-1)
    wgt = jnp.stack([w1, w2], axis=-1)
    return idx, wgt


def kernel(hidden_states, weight):
    bsz, seq_len, h = hidden_states.shape
    x = hidden_states.reshape(-1, h)
    topk_idx, topk_weight = _gate(x, weight)
    return (
        topk_idx.reshape(bsz, seq_len, -1),
        topk_weight.reshape(bsz, seq_len, -1),
    )
